# Initial kernel scaffold; baseline (speedup 1.0000x reference)
import functools

import jax
import jax.numpy as jnp
from jax import lax
from jax.experimental import pallas as pl
from jax.experimental.pallas import tpu as pltpu
from jax.experimental.pallas import tpu_sc as plsc

MAXZ = 84
NRP = 24
NATOMS = 1048576

NC = 2
NS = 16
NW = NC * NS

PER_W = NATOMS // NW
CHUNK = 2048
NCHUNK = PER_W // CHUNK


def _gather_sc(Z, p):
    mesh = plsc.VectorSubcoreMesh(core_axis_name="c", subcore_axis_name="s")

    @functools.partial(
        pl.kernel,
        mesh=mesh,
        out_type=jax.ShapeDtypeStruct((NATOMS, NRP), jnp.float32),
        scratch_types=[
            pltpu.VMEM((MAXZ, NRP), jnp.float32),
            pltpu.VMEM((CHUNK,), jnp.int32),
            pltpu.VMEM((CHUNK, NRP), jnp.float32),
            pltpu.SemaphoreType.DMA,
        ],
    )
    def k(z_hbm, p_hbm, out_hbm, table_v, idx_v, rows_v, sem):
        wid = lax.axis_index("s") * NC + lax.axis_index("c")
        base = wid * PER_W
        pltpu.sync_copy(p_hbm, table_v)

        def body(c, carry):
            off = base + c * CHUNK
            pltpu.sync_copy(z_hbm.at[pl.ds(off, CHUNK)], idx_v)
            pltpu.async_copy(table_v.at[idx_v], rows_v, sem).wait()
            pltpu.sync_copy(rows_v, out_hbm.at[pl.ds(off, CHUNK), :])
            return carry

        lax.fori_loop(0, NCHUNK, body, 0)

    return k(Z, p)


def kernel(Z, p, alpha, chi):
    Z32 = Z.astype(jnp.int32)
    gathered = _gather_sc(Z32, p)
    return (gathered, alpha, chi)


# speedup vs baseline: 6.1400x; 6.1400x over previous
import functools

import jax
import jax.numpy as jnp
from jax import lax
from jax.experimental import pallas as pl
from jax.experimental.pallas import tpu as pltpu
from jax.experimental.pallas import tpu_sc as plsc

MAXZ = 84
NRP = 24
NATOMS = 1048576

NC = 2
NS = 16
NW = NC * NS

PER_W = NATOMS // NW
CHUNK = 2048
NCHUNK = PER_W // CHUNK


def _gather_sc(Z, p):
    mesh = plsc.VectorSubcoreMesh(core_axis_name="c", subcore_axis_name="s")

    @functools.partial(
        pl.kernel,
        mesh=mesh,
        out_type=jax.ShapeDtypeStruct((NATOMS, NRP), jnp.float32),
        scratch_types=[
            pltpu.VMEM_SHARED((MAXZ, NRP), jnp.float32),
            pltpu.VMEM((CHUNK,), jnp.int32),
            pltpu.VMEM((CHUNK, NRP), jnp.float32),
            pltpu.SemaphoreType.DMA,
        ],
        compiler_params=pltpu.CompilerParams(use_tc_tiling_on_sc=False),
    )
    def k(z_hbm, p_hbm, out_hbm, table_v, idx_v, rows_v, sem):
        sid = lax.axis_index("s")
        wid = sid * NC + lax.axis_index("c")
        base = wid * PER_W

        @pl.when(sid == 0)
        def _stage():
            pltpu.sync_copy(p_hbm, table_v)

        plsc.subcore_barrier()

        def body(c, carry):
            off = base + c * CHUNK
            pltpu.sync_copy(z_hbm.at[pl.ds(off, CHUNK)], idx_v)
            pltpu.async_copy(table_v.at[idx_v], rows_v, sem).wait()
            pltpu.sync_copy(rows_v, out_hbm.at[pl.ds(off, CHUNK), :])
            return carry

        lax.fori_loop(0, NCHUNK, body, 0)

    return k(Z, p)


def kernel(Z, p, alpha, chi):
    Z32 = Z.astype(jnp.int32)
    gathered = _gather_sc(Z32, p)
    return (gathered, alpha, chi)
